# baseline (speedup 1.0000x reference)
import functools

import jax
import jax.numpy as jnp
from jax import lax
from jax.experimental import pallas as pl
from jax.experimental.pallas import tpu as pltpu
from jax.experimental.pallas import tpu_sc as plsc

_T = 64
_W = 3 * 224 * 224
_NC = 2
_NS = 16
_NW = _NC * _NS
_RPW = _T // _NW
_NCH = 8
_CH = _W // _NCH
_TOTAL = _RPW * _NCH
_NBUF = 6


def _sc_body(vid, out, *refs):
    bufs = refs[:_NBUF]
    gsems = refs[_NBUF:2 * _NBUF]
    wsems = refs[2 * _NBUF:3 * _NBUF]
    wid = lax.axis_index("s") * _NC + lax.axis_index("c")
    base_row = wid * _RPW

    def in_off(q):
        r = base_row + q // _NCH
        return r * _W + (q % _NCH) * _CH

    def out_off(q, dup):
        r = base_row + q // _NCH
        return (2 * r + dup) * _W + (q % _NCH) * _CH

    gh = [None] * _NBUF
    wh = [None] * _NBUF

    def issue_gather(q):
        b = q % _NBUF
        gh[b] = pltpu.async_copy(vid.at[pl.ds(in_off(q), _CH)], bufs[b],
                                 gsems[b])

    for q in range(min(_NBUF, _TOTAL)):
        issue_gather(q)
    for q in range(_TOTAL):
        b = q % _NBUF
        gh[b].wait()
        wh[b] = (
            pltpu.async_copy(bufs[b], out.at[pl.ds(out_off(q, 0), _CH)],
                             wsems[b]),
            pltpu.async_copy(bufs[b], out.at[pl.ds(out_off(q, 1), _CH)],
                             wsems[b]),
        )
        nxt = q + _NBUF
        if nxt < _TOTAL:
            wh[b][0].wait()
            wh[b][1].wait()
            issue_gather(nxt)
    for q in range(max(0, _TOTAL - _NBUF), _TOTAL):
        b = q % _NBUF
        wh[b][0].wait()
        wh[b][1].wait()


_sc_copy = functools.partial(
    pl.kernel,
    out_type=jax.ShapeDtypeStruct((2 * _T * _W,), jnp.float32),
    mesh=plsc.VectorSubcoreMesh(core_axis_name="c", subcore_axis_name="s"),
    scratch_types=(
        [pltpu.VMEM((_CH,), jnp.float32) for _ in range(_NBUF)]
        + [pltpu.SemaphoreType.DMA for _ in range(2 * _NBUF)]
    ),
)(_sc_body)


def kernel(video):
    vid = video.reshape(_T * _W)
    out = _sc_copy(vid)
    return out.reshape(2 * _T, 3, 224, 224)
